# Initial kernel scaffold; baseline (speedup 1.0000x reference)
import jax
import jax.numpy as jnp
from jax.experimental import pallas as pl


def kernel(ct_feats, tmpl_feats, s_feats):
    raise NotImplementedError("write your pallas kernel here")


# speedup vs baseline: 1.6451x; 1.6451x over previous
import jax
import jax.numpy as jnp
from jax import lax
from jax.experimental import pallas as pl
from jax.experimental.pallas import tpu as pltpu

_C = 768
_HW = 3136
_NP = 3584
_BQ = 512
_BK = 896
_NQB = _NP // _BQ
_NKB = _NP // _BK


def _fused_body(ct_ref, tm_ref, s_ref, out_ref, m_ref, p_ref, acc_ref):
    q = pl.program_id(0)
    k = pl.program_id(1)

    a = ct_ref[...]
    an2 = jnp.sum(a * a, axis=0, keepdims=True)
    a1 = a / (jnp.sqrt(an2 + 1e-8) + 1e-8)
    a2 = a / (jnp.sqrt(an2) + 1e-8)

    b = tm_ref[...]
    bn2 = jnp.sum(b * b, axis=0, keepdims=True)
    b1 = b / (jnp.sqrt(bn2 + 1e-8) + 1e-8)

    s = s_ref[...]
    sn2 = jnp.sum(s * s, axis=0, keepdims=True)
    s2 = s / (jnp.sqrt(sn2) + 1e-8)

    dn = (((0,), (0,)), ((), ()))
    sim = lax.dot_general(a1, b1, dn, preferred_element_type=jnp.float32)
    pval = lax.dot_general(a2, s2, dn, preferred_element_type=jnp.float32)

    col = k * _BK + lax.broadcasted_iota(jnp.int32, (_BQ, _BK), 1)
    sim = jnp.where(col < _HW, sim, -3e38)
    m_blk = jnp.max(sim, axis=1, keepdims=True)
    am = jnp.min(jnp.where(sim == m_blk, col, jnp.int32(2**30)),
                 axis=1, keepdims=True)
    p_blk = jnp.sum(jnp.where(col == am, pval, 0.0), axis=1, keepdims=True)

    @pl.when(k == 0)
    def _():
        m_ref[...] = m_blk
        p_ref[...] = p_blk

    @pl.when(k > 0)
    def _():
        upd = m_blk > m_ref[...]
        p_ref[...] = jnp.where(upd, p_blk, p_ref[...])
        m_ref[...] = jnp.where(upd, m_blk, m_ref[...])

    @pl.when(jnp.logical_and(q == 0, k == 0))
    def _():
        acc_ref[0, 0] = 0.0

    @pl.when(k == _NKB - 1)
    def _():
        acc_ref[0, 0] += jnp.sum(p_ref[...])

    @pl.when(jnp.logical_and(q == _NQB - 1, k == _NKB - 1))
    def _():
        out_ref[0, 0] = 1.0 - acc_ref[0, 0] / _HW


def kernel(ct_feats, tmpl_feats, s_feats):
    n, c, h, w = ct_feats.shape
    hw = h * w
    pad = _NP - hw
    ct = jnp.pad(ct_feats.reshape(c, hw), ((0, 0), (0, pad)))
    tm = jnp.pad(tmpl_feats.reshape(c, hw), ((0, 0), (0, pad)))
    s = jnp.pad(s_feats.reshape(c, hw), ((0, 0), (0, pad)))

    out = pl.pallas_call(
        _fused_body,
        grid=(_NQB, _NKB),
        in_specs=[
            pl.BlockSpec((_C, _BQ), lambda q, k: (0, q)),
            pl.BlockSpec((_C, _BK), lambda q, k: (0, k)),
            pl.BlockSpec((_C, _BK), lambda q, k: (0, k)),
        ],
        out_specs=pl.BlockSpec(memory_space=pltpu.SMEM),
        out_shape=jax.ShapeDtypeStruct((1, 1), jnp.float32),
        scratch_shapes=[
            pltpu.VMEM((_BQ, 1), jnp.float32),
            pltpu.VMEM((_BQ, 1), jnp.float32),
            pltpu.SMEM((1, 1), jnp.float32),
        ],
    )(ct, tm, s)
    return out[0, 0]


# speedup vs baseline: 1.7594x; 1.0695x over previous
import functools

import jax
import jax.numpy as jnp
from jax import lax
from jax.experimental import pallas as pl
from jax.experimental.pallas import tpu as pltpu
from jax.experimental.pallas import tpu_sc as plsc

_C = 768
_HW = 3136
_NP = 3584
_BQ = 512
_BK = 896
_NQB = _NP // _BQ
_NKB = _NP // _BK

_NW = 32
_BPW = _NP // _NW


def _argmin_body(ct_ref, tm_ref, z_ref, m_ref, zb_ref):
    k = pl.program_id(1)

    a = ct_ref[...]
    an2 = jnp.sum(a * a, axis=1, keepdims=True)
    a1 = a / (jnp.sqrt(an2 + 1e-8) + 1e-8)

    b = tm_ref[...]
    bn2 = jnp.sum(b * b, axis=0, keepdims=True)
    b1 = b / (jnp.sqrt(bn2 + 1e-8) + 1e-8)

    sim = lax.dot_general(b1, a1, (((0,), (1,)), ((), ())),
                          preferred_element_type=jnp.float32)
    col = k * _BK + lax.broadcasted_iota(jnp.int32, (_BK, _BQ), 0)
    sim = jnp.where(col < _HW, sim, -3e38)
    m_blk = jnp.max(sim, axis=0, keepdims=True)
    z_blk = jnp.min(jnp.where(sim == m_blk, col, jnp.int32(2**30)),
                    axis=0, keepdims=True)

    @pl.when(k == 0)
    def _():
        m_ref[...] = m_blk
        zb_ref[...] = z_blk

    @pl.when(k > 0)
    def _():
        upd = m_blk > m_ref[...]
        zb_ref[...] = jnp.where(upd, z_blk, zb_ref[...])
        m_ref[...] = jnp.where(upd, m_blk, m_ref[...])

    @pl.when(k == _NKB - 1)
    def _():
        z_ref[0] = zb_ref[...]


@functools.cache
def _sc_gather_fn():
    mesh = plsc.VectorSubcoreMesh(core_axis_name="c", subcore_axis_name="s")

    @functools.partial(
        pl.kernel,
        mesh=mesh,
        out_type=jax.ShapeDtypeStruct((_NP, _C), jnp.float32),
        scratch_types=[
            pltpu.VMEM((_BPW,), jnp.int32),
            pltpu.VMEM((_BPW, _C), jnp.float32),
            pltpu.SemaphoreType.DMA,
        ],
    )
    def _sc_gather(s_hbm, idx_hbm, out_hbm, idx_v, rows_v, sem):
        wid = lax.axis_index("s") * 2 + lax.axis_index("c")
        base = wid * _BPW
        pltpu.sync_copy(idx_hbm.at[pl.ds(base, _BPW)], idx_v)
        pltpu.async_copy(s_hbm.at[idx_v], rows_v, sem).wait()
        pltpu.sync_copy(rows_v, out_hbm.at[pl.ds(base, _BPW)])

    return _sc_gather


def _loss_body(ct_ref, g_ref, out_ref, acc_ref):
    q = pl.program_id(0)

    a = ct_ref[...]
    g = g_ref[...]
    an2 = jnp.sum(a * a, axis=1, keepdims=True)
    gn2 = jnp.sum(g * g, axis=1, keepdims=True)
    dots = jnp.sum(a * g, axis=1, keepdims=True)
    p = dots / ((jnp.sqrt(an2) + 1e-8) * (jnp.sqrt(gn2) + 1e-8))

    @pl.when(q == 0)
    def _():
        acc_ref[0, 0] = 0.0

    acc_ref[0, 0] += jnp.sum(p)

    @pl.when(q == _NQB - 1)
    def _():
        out_ref[0, 0] = 1.0 - acc_ref[0, 0] / _HW


def kernel(ct_feats, tmpl_feats, s_feats):
    n, c, h, w = ct_feats.shape
    hw = h * w
    pad = _NP - hw
    ct_t = jnp.pad(ct_feats.reshape(c, hw).T, ((0, pad), (0, 0)))
    tm = jnp.pad(tmpl_feats.reshape(c, hw), ((0, 0), (0, pad)))
    s_t = s_feats.reshape(c, hw).T

    z = pl.pallas_call(
        _argmin_body,
        grid=(_NQB, _NKB),
        in_specs=[
            pl.BlockSpec((_BQ, _C), lambda q, k: (q, 0)),
            pl.BlockSpec((_C, _BK), lambda q, k: (0, k)),
        ],
        out_specs=pl.BlockSpec((1, 1, _BQ), lambda q, k: (q, 0, 0)),
        out_shape=jax.ShapeDtypeStruct((_NQB, 1, _BQ), jnp.int32),
        scratch_shapes=[
            pltpu.VMEM((1, _BQ), jnp.float32),
            pltpu.VMEM((1, _BQ), jnp.int32),
        ],
    )(ct_t, tm)

    feat = _sc_gather_fn()(s_t, z.reshape(_NP))

    out = pl.pallas_call(
        _loss_body,
        grid=(_NQB,),
        in_specs=[
            pl.BlockSpec((_BQ, _C), lambda q: (q, 0)),
            pl.BlockSpec((_BQ, _C), lambda q: (q, 0)),
        ],
        out_specs=pl.BlockSpec(memory_space=pltpu.SMEM),
        out_shape=jax.ShapeDtypeStruct((1, 1), jnp.float32),
        scratch_shapes=[pltpu.SMEM((1, 1), jnp.float32)],
    )(ct_t, feat)
    return out[0, 0]


# speedup vs baseline: 1.7805x; 1.0120x over previous
import functools

import jax
import jax.numpy as jnp
from jax import lax
from jax.experimental import pallas as pl
from jax.experimental.pallas import tpu as pltpu
from jax.experimental.pallas import tpu_sc as plsc

_C = 768
_HW = 3136
_NP = 3584
_BQ = 512
_BK = 896
_NQB = _NP // _BQ
_NKB = _NP // _BK

_NW = 32
_BPW = _NP // _NW


def _argmin_body(ct_ref, tm_ref, z_ref, m_ref, zb_ref):
    k = pl.program_id(1)

    a = ct_ref[...]
    an2 = jnp.sum(a * a, axis=1, keepdims=True)
    a1 = a / (jnp.sqrt(an2 + 1e-8) + 1e-8)

    b = tm_ref[...]
    bn2 = jnp.sum(b * b, axis=0, keepdims=True)
    b1 = b / (jnp.sqrt(bn2 + 1e-8) + 1e-8)

    sim = lax.dot_general(b1.astype(jnp.bfloat16), a1.astype(jnp.bfloat16),
                          (((0,), (1,)), ((), ())),
                          preferred_element_type=jnp.float32)
    col = k * _BK + lax.broadcasted_iota(jnp.int32, (_BK, _BQ), 0)
    sim = jnp.where(col < _HW, sim, -3e38)
    m_blk = jnp.max(sim, axis=0, keepdims=True)
    z_blk = jnp.min(jnp.where(sim == m_blk, col, jnp.int32(2**30)),
                    axis=0, keepdims=True)

    @pl.when(k == 0)
    def _():
        m_ref[...] = m_blk
        zb_ref[...] = z_blk

    @pl.when(k > 0)
    def _():
        upd = m_blk > m_ref[...]
        zb_ref[...] = jnp.where(upd, z_blk, zb_ref[...])
        m_ref[...] = jnp.where(upd, m_blk, m_ref[...])

    @pl.when(k == _NKB - 1)
    def _():
        z_ref[0] = zb_ref[...]


@functools.cache
def _sc_gather_fn():
    mesh = plsc.VectorSubcoreMesh(core_axis_name="c", subcore_axis_name="s")

    @functools.partial(
        pl.kernel,
        mesh=mesh,
        out_type=jax.ShapeDtypeStruct((_NP, _C), jnp.float32),
        scratch_types=[
            pltpu.VMEM((_BPW,), jnp.int32),
            pltpu.VMEM((_BPW, _C), jnp.float32),
            pltpu.SemaphoreType.DMA,
        ],
    )
    def _sc_gather(s_hbm, idx_hbm, out_hbm, idx_v, rows_v, sem):
        wid = lax.axis_index("s") * 2 + lax.axis_index("c")
        base = wid * _BPW
        pltpu.sync_copy(idx_hbm.at[pl.ds(base, _BPW)], idx_v)
        pltpu.async_copy(s_hbm.at[idx_v], rows_v, sem).wait()
        pltpu.sync_copy(rows_v, out_hbm.at[pl.ds(base, _BPW)])

    return _sc_gather


def _loss_body(ct_ref, g_ref, out_ref, acc_ref):
    q = pl.program_id(0)

    a = ct_ref[...]
    g = g_ref[...]
    an2 = jnp.sum(a * a, axis=1, keepdims=True)
    gn2 = jnp.sum(g * g, axis=1, keepdims=True)
    dots = jnp.sum(a * g, axis=1, keepdims=True)
    p = dots / ((jnp.sqrt(an2) + 1e-8) * (jnp.sqrt(gn2) + 1e-8))

    @pl.when(q == 0)
    def _():
        acc_ref[0, 0] = 0.0

    acc_ref[0, 0] += jnp.sum(p)

    @pl.when(q == _NQB - 1)
    def _():
        out_ref[0, 0] = 1.0 - acc_ref[0, 0] / _HW


def kernel(ct_feats, tmpl_feats, s_feats):
    n, c, h, w = ct_feats.shape
    hw = h * w
    pad = _NP - hw
    ct_t = jnp.pad(ct_feats.reshape(c, hw).T, ((0, pad), (0, 0)))
    tm = jnp.pad(tmpl_feats.reshape(c, hw), ((0, 0), (0, pad)))
    s_t = s_feats.reshape(c, hw).T

    z = pl.pallas_call(
        _argmin_body,
        grid=(_NQB, _NKB),
        in_specs=[
            pl.BlockSpec((_BQ, _C), lambda q, k: (q, 0)),
            pl.BlockSpec((_C, _BK), lambda q, k: (0, k)),
        ],
        out_specs=pl.BlockSpec((1, 1, _BQ), lambda q, k: (q, 0, 0)),
        out_shape=jax.ShapeDtypeStruct((_NQB, 1, _BQ), jnp.int32),
        scratch_shapes=[
            pltpu.VMEM((1, _BQ), jnp.float32),
            pltpu.VMEM((1, _BQ), jnp.int32),
        ],
    )(ct_t, tm)

    feat = _sc_gather_fn()(s_t, z.reshape(_NP))

    out = pl.pallas_call(
        _loss_body,
        grid=(_NQB,),
        in_specs=[
            pl.BlockSpec((_BQ, _C), lambda q: (q, 0)),
            pl.BlockSpec((_BQ, _C), lambda q: (q, 0)),
        ],
        out_specs=pl.BlockSpec(memory_space=pltpu.SMEM),
        out_shape=jax.ShapeDtypeStruct((1, 1), jnp.float32),
        scratch_shapes=[pltpu.SMEM((1, 1), jnp.float32)],
    )(ct_t, feat)
    return out[0, 0]
